# baseline (speedup 1.0000x reference)
import functools

import jax
import jax.numpy as jnp
from jax import lax
from jax.experimental import pallas as pl
from jax.experimental.pallas import tpu as pltpu
from jax.experimental.pallas import tpu_sc as plsc

N = 4096
D = 16
DP = 128
BQ = 1024
GRID = N // BQ
INF = float("inf")


def _nn_idx_body(x_ref, idx_ref, xpad_ref, x2_ref):
    i = pl.program_id(0)

    @pl.when(i == 0)
    def _build_x2():
        xf = x_ref[...]
        x2 = jnp.sum(xf * xf, axis=1, keepdims=True)
        x2_ref[...] = x2.reshape(1, N)

    q = x_ref[pl.ds(i * BQ, BQ), :]
    xpad_ref[:, :D] = q
    g = lax.dot_general(q, x_ref[...], (((1,), (1,)), ((), ())),
                        preferred_element_type=jnp.float32)
    q2 = jnp.sum(q * q, axis=1, keepdims=True)
    G8 = BQ // 8
    s1 = (q2.reshape(G8, 8, 1) + x2_ref[...].reshape(1, 1, N))
    d2 = (s1 - 2.0 * g.reshape(G8, 8, N)).reshape(BQ, N)

    cols = lax.broadcasted_iota(jnp.int32, (BQ, N), 1)
    rows = i * BQ + lax.broadcasted_iota(jnp.int32, (BQ, N), 0)
    d2 = jnp.where(cols == rows, INF, d2)

    m = jnp.min(d2, axis=1, keepdims=True)
    idx = jnp.min(jnp.where(d2 <= m, cols, 2 * N), axis=1)
    idx_ref[...] = idx.reshape(BQ // 128, 128)


def _nn_indices(x):
    return pl.pallas_call(
        _nn_idx_body,
        grid=(GRID,),
        in_specs=[
            pl.BlockSpec((N, D), lambda i: (0, 0)),
        ],
        out_specs=[
            pl.BlockSpec((BQ // 128, 128), lambda i: (i, 0)),
            pl.BlockSpec((BQ, DP), lambda i: (i, 0)),
        ],
        out_shape=[
            jax.ShapeDtypeStruct((N // 128, 128), jnp.int32),
            jax.ShapeDtypeStruct((N, DP), jnp.float32),
        ],
        scratch_shapes=[pltpu.VMEM((1, N), jnp.float32)],
    )(x)


def _make_sc_gather():
    info = plsc.get_sparse_core_info()
    nw = info.num_cores * info.num_subcores
    b_per_w = N // nw
    mesh = plsc.VectorSubcoreMesh(core_axis_name="c", subcore_axis_name="s")

    @functools.partial(
        pl.kernel,
        mesh=mesh,
        compiler_params=pltpu.CompilerParams(skip_device_barrier=True),
        out_type=jax.ShapeDtypeStruct((N, DP), jnp.float32),
        scratch_types=[
            pltpu.VMEM((8, 128), jnp.int32),
            pltpu.VMEM((b_per_w, DP), jnp.float32),
            pltpu.SemaphoreType.DMA,
        ],
    )
    def gather(table_hbm, idx_hbm, out_hbm, idx_v, rows_v, sem):
        wid = lax.axis_index("s") * info.num_cores + lax.axis_index("c")
        pltpu.sync_copy(idx_hbm.at[pl.ds(8 * (wid // 8), 8)], idx_v)
        pltpu.async_copy(table_hbm.at[idx_v.at[wid % 8]], rows_v, sem).wait()
        pltpu.sync_copy(rows_v, out_hbm.at[pl.ds(wid * b_per_w, b_per_w)])

    return gather


_sc_gather = None


def kernel(x, queue_buf):
    del queue_buf
    global _sc_gather
    if _sc_gather is None:
        _sc_gather = _make_sc_gather()
    idxm, xpad = _nn_indices(x)
    return _sc_gather(xpad, idxm)[:, :D]


# speedup vs baseline: 1.0246x; 1.0246x over previous
import functools

import jax
import jax.numpy as jnp
from jax import lax
from jax.experimental import pallas as pl
from jax.experimental.pallas import tpu as pltpu
from jax.experimental.pallas import tpu_sc as plsc

N = 4096
D = 16
DP = 128
BQ = 2048
GRID = N // BQ
INF = float("inf")


def _nn_idx_body(x_ref, idx_ref, xpad_ref, x2_ref):
    i = pl.program_id(0)

    @pl.when(i == 0)
    def _build_x2():
        xf = x_ref[...]
        x2 = jnp.sum(xf * xf, axis=1, keepdims=True)
        x2_ref[...] = x2.reshape(1, N)

    q = x_ref[pl.ds(i * BQ, BQ), :]
    xpad_ref[:, :D] = q
    g = lax.dot_general(q, x_ref[...], (((1,), (1,)), ((), ())),
                        preferred_element_type=jnp.float32)
    q2 = jnp.sum(q * q, axis=1, keepdims=True)
    G8 = BQ // 8
    s1 = (q2.reshape(G8, 8, 1) + x2_ref[...].reshape(1, 1, N))
    d2 = (s1 - 2.0 * g.reshape(G8, 8, N)).reshape(BQ, N)

    cols = lax.broadcasted_iota(jnp.int32, (BQ, N), 1)
    rows = i * BQ + lax.broadcasted_iota(jnp.int32, (BQ, N), 0)
    d2 = jnp.where(cols == rows, INF, d2)

    m = jnp.min(d2, axis=1, keepdims=True)
    idx = jnp.min(jnp.where(d2 <= m, cols, 2 * N), axis=1)
    idx_ref[...] = idx.reshape(BQ // 128, 128)


def _nn_indices(x):
    return pl.pallas_call(
        _nn_idx_body,
        grid=(GRID,),
        in_specs=[
            pl.BlockSpec((N, D), lambda i: (0, 0)),
        ],
        out_specs=[
            pl.BlockSpec((BQ // 128, 128), lambda i: (i, 0)),
            pl.BlockSpec((BQ, DP), lambda i: (i, 0)),
        ],
        out_shape=[
            jax.ShapeDtypeStruct((N // 128, 128), jnp.int32),
            jax.ShapeDtypeStruct((N, DP), jnp.float32),
        ],
        scratch_shapes=[pltpu.VMEM((1, N), jnp.float32)],
        compiler_params=pltpu.CompilerParams(vmem_limit_bytes=110 * 1024 * 1024),
    )(x)


def _make_sc_gather():
    info = plsc.get_sparse_core_info()
    nw = info.num_cores * info.num_subcores
    b_per_w = N // nw
    mesh = plsc.VectorSubcoreMesh(core_axis_name="c", subcore_axis_name="s")

    @functools.partial(
        pl.kernel,
        mesh=mesh,
        compiler_params=pltpu.CompilerParams(skip_device_barrier=True),
        out_type=jax.ShapeDtypeStruct((N, DP), jnp.float32),
        scratch_types=[
            pltpu.VMEM((8, 128), jnp.int32),
            pltpu.VMEM((b_per_w, DP), jnp.float32),
            pltpu.SemaphoreType.DMA,
        ],
    )
    def gather(table_hbm, idx_hbm, out_hbm, idx_v, rows_v, sem):
        wid = lax.axis_index("s") * info.num_cores + lax.axis_index("c")
        pltpu.sync_copy(idx_hbm.at[pl.ds(8 * (wid // 8), 8)], idx_v)
        pltpu.async_copy(table_hbm.at[idx_v.at[wid % 8]], rows_v, sem).wait()
        pltpu.sync_copy(rows_v, out_hbm.at[pl.ds(wid * b_per_w, b_per_w)])

    return gather


_sc_gather = None


def kernel(x, queue_buf):
    del queue_buf
    global _sc_gather
    if _sc_gather is None:
        _sc_gather = _make_sc_gather()
    idxm, xpad = _nn_indices(x)
    return _sc_gather(xpad, idxm)[:, :D]
